# baseline (speedup 1.0000x reference)
import jax
import jax.numpy as jnp
from jax import lax
from jax.experimental import pallas as pl
from jax.experimental.pallas import tpu as pltpu
from jax.experimental.pallas import tpu_sc as plsc

N_NODES = 10000
N_PAD = 10240
D = 128
N_EDGES = 320000
CHUNK = 128
N_CHUNKS = N_EDGES // CHUNK
NS = 16
ROWS_PER_TILE = N_PAD // NS
CH_BASE = N_CHUNKS // NS
CH_REM = N_CHUNKS % NS


def _sc_body(xa_hbm, idx_hbm, zeros2_hbm, zeros1_hbm, feat_out, deg_out,
             idx_v, rows_v, hist_v, acc_sh, gsem0, gsem1, ssem0, ssem1):
    c = lax.axis_index("c")
    s = lax.axis_index("s")
    pltpu.sync_copy(zeros2_hbm, acc_sh.at[pl.ds(s * ROWS_PER_TILE, ROWS_PER_TILE)])
    pltpu.sync_copy(zeros1_hbm, hist_v)

    lo = s * CH_BASE + jnp.minimum(s, CH_REM)
    n = CH_BASE + jnp.where(s < CH_REM, 1, 0)
    ones16 = jnp.ones((16,), jnp.float32)
    gsems = (gsem0, gsem1)
    ssems = (ssem0, ssem1)

    def hist_chunk(q):
        for k in range(CHUNK // 16):
            idx16 = idx_v[q, 1, pl.ds(k * 16, 16)]
            plsc.addupdate_scatter(hist_v, [idx16], ones16)

    def stage(j, q):
        pltpu.sync_copy(idx_hbm.at[c, j], idx_v.at[q])

    def start_gather(q):
        pltpu.async_copy(xa_hbm.at[idx_v.at[q, 0]], rows_v.at[q & 1],
                         gsems[q & 1])

    def start_scatter(q):
        pltpu.async_copy(rows_v.at[q & 1], acc_sh.at[idx_v.at[q, 1]],
                         ssems[q & 1], add=True)

    def wait_scatter(q):
        pltpu.make_async_copy(rows_v.at[q & 1], acc_sh.at[idx_v.at[q, 1]],
                              ssems[q & 1]).wait()

    def drain(q):
        pltpu.make_async_copy(xa_hbm.at[idx_v.at[q, 0]], rows_v.at[q & 1],
                              gsems[q & 1]).wait()
        start_scatter(q)
        hist_chunk(q)

    for qq in range(4):
        @pl.when((lo & 3) == qq)
        def _(qq=qq):
            stage(lo, qq)
            start_gather(qq)

    plsc.subcore_barrier()

    for qq in range(4):
        @pl.when(((lo + 1) & 3) == qq)
        def _(qq=qq):
            stage(lo + 1, qq)
            start_gather(qq)
            drain((qq - 1) & 3)

    def body(j, carry):
        q = j & 3
        for qq in range(4):
            @pl.when(q == qq)
            def _(qq=qq):
                wait_scatter((qq + 2) & 3)
                stage(j, qq)
                start_gather(qq)
                drain((qq - 1) & 3)
        return carry

    lax.fori_loop(lo + 2, lo + n, body, 0)

    h = lo + n - 1
    for qq in range(4):
        @pl.when((h & 3) == qq)
        def _(qq=qq):
            drain(qq)
            wait_scatter((qq - 1) & 3)
            wait_scatter(qq)

    pltpu.sync_copy(hist_v, deg_out.at[c, s])
    plsc.subcore_barrier()
    pltpu.sync_copy(acc_sh.at[pl.ds(s * ROWS_PER_TILE, ROWS_PER_TILE)],
                    feat_out.at[c, pl.ds(s * ROWS_PER_TILE, ROWS_PER_TILE)])


_sc_call = pl.kernel(
    _sc_body,
    out_type=(
        jax.ShapeDtypeStruct((2, N_PAD, D), jnp.float32),
        jax.ShapeDtypeStruct((2, NS, N_PAD), jnp.float32),
    ),
    mesh=plsc.VectorSubcoreMesh(core_axis_name="c", subcore_axis_name="s"),
    compiler_params=pltpu.CompilerParams(needs_layout_passes=False),
    scratch_types=[
        pltpu.VMEM((4, 2, CHUNK), jnp.int32),
        pltpu.VMEM((2, CHUNK, D), jnp.float32),
        pltpu.VMEM((N_PAD,), jnp.float32),
        pltpu.VMEM_SHARED((N_PAD, D), jnp.float32),
        pltpu.SemaphoreType.DMA,
        pltpu.SemaphoreType.DMA,
        pltpu.SemaphoreType.DMA,
        pltpu.SemaphoreType.DMA,
    ],
)


BLK = 1024


def _tc_body(x_ref, aF_ref, dF_ref, aR_ref, dR_ref,
             wl1_ref, wr1_ref, wl2_ref, wr2_ref, b1_ref, b2_ref, out_ref):
    x = x_ref[...]
    ones_col = jnp.ones((NS, 1), jnp.float32)
    dn = (((0,), (0,)), ((), ()))
    degF = lax.dot_general(dF_ref[...], ones_col, dn,
                           preferred_element_type=jnp.float32)
    degR = lax.dot_general(dR_ref[...], ones_col, dn,
                           preferred_element_type=jnp.float32)
    meanF = aF_ref[...] * (1.0 / jnp.maximum(degF, 1.0))
    meanR = aR_ref[...] * (1.0 / jnp.maximum(degR, 1.0))
    outF = (jnp.dot(meanF, wl1_ref[...], preferred_element_type=jnp.float32)
            + b1_ref[...]
            + jnp.dot(x, wr1_ref[...], preferred_element_type=jnp.float32))
    outR = (jnp.dot(meanR, wl2_ref[...], preferred_element_type=jnp.float32)
            + b2_ref[...]
            + jnp.dot(x, wr2_ref[...], preferred_element_type=jnp.float32))
    out_ref[:, :D] = outF
    out_ref[:, D:] = outR


def _tc_call(x, aF, dFt, aR, dRt, wl1t, wr1t, wl2t, wr2t, b1, b2):
    grid = pl.cdiv(N_NODES, BLK)
    row_spec = pl.BlockSpec((BLK, D), lambda i: (i, 0))
    deg_spec = pl.BlockSpec((NS, BLK), lambda i: (0, i))
    full_spec = lambda a, b: pl.BlockSpec((a, b), lambda i: (0, 0))
    return pl.pallas_call(
        _tc_body,
        grid=(grid,),
        in_specs=[
            row_spec, row_spec, deg_spec, row_spec, deg_spec,
            full_spec(D, D), full_spec(D, D), full_spec(D, D), full_spec(D, D),
            full_spec(1, D), full_spec(1, D),
        ],
        out_specs=pl.BlockSpec((BLK, 2 * D), lambda i: (i, 0)),
        out_shape=jax.ShapeDtypeStruct((N_NODES, 2 * D), jnp.float32),
    )(x, aF, dFt, aR, dRt, wl1t, wr1t, wl2t, wr2t, b1, b2)


@jax.jit
def kernel(x, edge_index, W_l1, b_l1, W_r1, W_l2, b_l2, W_r2):
    ei = edge_index.astype(jnp.int32)
    src, dst = ei[0], ei[1]
    fwd = jnp.stack([src.reshape(N_CHUNKS, CHUNK), dst.reshape(N_CHUNKS, CHUNK)], 1)
    rev = fwd[:, ::-1]
    idx = jnp.stack([fwd, rev])
    zeros2 = jnp.zeros((ROWS_PER_TILE, D), jnp.float32)
    zeros1 = jnp.zeros((N_PAD,), jnp.float32)
    feat, deg = _sc_call(x, idx, zeros2, zeros1)
    aF = feat[0, :N_NODES]
    aR = feat[1, :N_NODES]
    return _tc_call(x, aF, deg[0], aR, deg[1],
                    W_l1.T, W_r1.T, W_l2.T, W_r2.T,
                    b_l1.reshape(1, D), b_l2.reshape(1, D))


# speedup vs baseline: 1.1336x; 1.1336x over previous
import jax
import jax.numpy as jnp
from jax import lax
from jax.experimental import pallas as pl
from jax.experimental.pallas import tpu as pltpu
from jax.experimental.pallas import tpu_sc as plsc

N_NODES = 10000
N_PAD = 10240
D = 128
N_EDGES = 320000
CHUNK = 128
N_CHUNKS = N_EDGES // CHUNK
NS = 16
ROWS_PER_TILE = N_PAD // NS
CH_BASE = N_CHUNKS // NS
CH_REM = N_CHUNKS % NS


def _sc_body(xa_hbm, idx_hbm, zeros2_hbm, zeros1_hbm, feat_out, deg_out,
             idx_v, rows_v, hist_v, acc_sh, gsem0, gsem1, ssem0, ssem1, isem):
    c = lax.axis_index("c")
    s = lax.axis_index("s")
    pltpu.sync_copy(zeros2_hbm, acc_sh.at[pl.ds(s * ROWS_PER_TILE, ROWS_PER_TILE)])
    pltpu.sync_copy(zeros1_hbm, hist_v)

    lo = s * CH_BASE + jnp.minimum(s, CH_REM)
    n = CH_BASE + jnp.where(s < CH_REM, 1, 0)
    ones16 = jnp.ones((16,), jnp.float32)
    gsems = (gsem0, gsem1)
    ssems = (ssem0, ssem1)

    def hist_chunk(q):
        for k in range(CHUNK // 16):
            idx16 = idx_v[q, 1, pl.ds(k * 16, 16)]
            plsc.addupdate_scatter(hist_v, [idx16], ones16)

    def stage(j, q):
        pltpu.sync_copy(idx_hbm.at[c, j], idx_v.at[q])

    def prefetch(j, q):
        pltpu.async_copy(idx_hbm.at[c, jnp.minimum(j, N_CHUNKS - 1)],
                         idx_v.at[q], isem)

    def wait_prefetch(j, q):
        pltpu.make_async_copy(idx_hbm.at[c, jnp.minimum(j, N_CHUNKS - 1)],
                              idx_v.at[q], isem).wait()

    def start_gather(q):
        pltpu.async_copy(xa_hbm.at[idx_v.at[q, 0]], rows_v.at[q & 1],
                         gsems[q & 1])

    def start_scatter(q):
        pltpu.async_copy(rows_v.at[q & 1], acc_sh.at[idx_v.at[q, 1]],
                         ssems[q & 1], add=True)

    def wait_scatter(q):
        pltpu.make_async_copy(rows_v.at[q & 1], acc_sh.at[idx_v.at[q, 1]],
                              ssems[q & 1]).wait()

    def drain(q):
        pltpu.make_async_copy(xa_hbm.at[idx_v.at[q, 0]], rows_v.at[q & 1],
                              gsems[q & 1]).wait()
        start_scatter(q)
        hist_chunk(q)

    for qq in range(4):
        @pl.when((lo & 3) == qq)
        def _(qq=qq):
            stage(lo, qq)
            start_gather(qq)
            prefetch(lo + 1, (qq + 1) & 3)

    plsc.subcore_barrier()

    for qq in range(4):
        @pl.when(((lo + 1) & 3) == qq)
        def _(qq=qq):
            wait_prefetch(lo + 1, qq)
            start_gather(qq)
            prefetch(lo + 2, (qq + 1) & 3)
            drain((qq - 1) & 3)

    def body(j, carry):
        q = j & 3
        for qq in range(4):
            @pl.when(q == qq)
            def _(qq=qq):
                wait_scatter((qq + 2) & 3)
                wait_prefetch(j, qq)
                start_gather(qq)
                prefetch(j + 1, (qq + 1) & 3)
                drain((qq - 1) & 3)
        return carry

    lax.fori_loop(lo + 2, lo + n, body, 0)

    h = lo + n - 1
    for qq in range(4):
        @pl.when((h & 3) == qq)
        def _(qq=qq):
            drain(qq)
            wait_scatter((qq - 1) & 3)
            wait_scatter(qq)
            wait_prefetch(h + 1, (qq + 1) & 3)

    pltpu.sync_copy(hist_v, deg_out.at[c, s])
    plsc.subcore_barrier()
    pltpu.sync_copy(acc_sh.at[pl.ds(s * ROWS_PER_TILE, ROWS_PER_TILE)],
                    feat_out.at[c, pl.ds(s * ROWS_PER_TILE, ROWS_PER_TILE)])


_sc_call = pl.kernel(
    _sc_body,
    out_type=(
        jax.ShapeDtypeStruct((2, N_PAD, D), jnp.float32),
        jax.ShapeDtypeStruct((2, NS, N_PAD), jnp.float32),
    ),
    mesh=plsc.VectorSubcoreMesh(core_axis_name="c", subcore_axis_name="s"),
    compiler_params=pltpu.CompilerParams(needs_layout_passes=False),
    scratch_types=[
        pltpu.VMEM((4, 2, CHUNK), jnp.int32),
        pltpu.VMEM((2, CHUNK, D), jnp.float32),
        pltpu.VMEM((N_PAD,), jnp.float32),
        pltpu.VMEM_SHARED((N_PAD, D), jnp.float32),
        pltpu.SemaphoreType.DMA,
        pltpu.SemaphoreType.DMA,
        pltpu.SemaphoreType.DMA,
        pltpu.SemaphoreType.DMA,
        pltpu.SemaphoreType.DMA,
    ],
)


BLK = 1024


def _tc_body(x_ref, aF_ref, dF_ref, aR_ref, dR_ref,
             wl1_ref, wr1_ref, wl2_ref, wr2_ref, b1_ref, b2_ref, out_ref):
    x = x_ref[...]
    ones_col = jnp.ones((NS, 1), jnp.float32)
    dn = (((0,), (0,)), ((), ()))
    degF = lax.dot_general(dF_ref[...], ones_col, dn,
                           preferred_element_type=jnp.float32)
    degR = lax.dot_general(dR_ref[...], ones_col, dn,
                           preferred_element_type=jnp.float32)
    meanF = aF_ref[...] * (1.0 / jnp.maximum(degF, 1.0))
    meanR = aR_ref[...] * (1.0 / jnp.maximum(degR, 1.0))
    outF = (jnp.dot(meanF, wl1_ref[...], preferred_element_type=jnp.float32)
            + b1_ref[...]
            + jnp.dot(x, wr1_ref[...], preferred_element_type=jnp.float32))
    outR = (jnp.dot(meanR, wl2_ref[...], preferred_element_type=jnp.float32)
            + b2_ref[...]
            + jnp.dot(x, wr2_ref[...], preferred_element_type=jnp.float32))
    out_ref[:, :D] = outF
    out_ref[:, D:] = outR


def _tc_call(x, aF, dFt, aR, dRt, wl1t, wr1t, wl2t, wr2t, b1, b2):
    grid = pl.cdiv(N_NODES, BLK)
    row_spec = pl.BlockSpec((BLK, D), lambda i: (i, 0))
    deg_spec = pl.BlockSpec((NS, BLK), lambda i: (0, i))
    full_spec = lambda a, b: pl.BlockSpec((a, b), lambda i: (0, 0))
    return pl.pallas_call(
        _tc_body,
        grid=(grid,),
        in_specs=[
            row_spec, row_spec, deg_spec, row_spec, deg_spec,
            full_spec(D, D), full_spec(D, D), full_spec(D, D), full_spec(D, D),
            full_spec(1, D), full_spec(1, D),
        ],
        out_specs=pl.BlockSpec((BLK, 2 * D), lambda i: (i, 0)),
        out_shape=jax.ShapeDtypeStruct((N_NODES, 2 * D), jnp.float32),
    )(x, aF, dFt, aR, dRt, wl1t, wr1t, wl2t, wr2t, b1, b2)


@jax.jit
def kernel(x, edge_index, W_l1, b_l1, W_r1, W_l2, b_l2, W_r2):
    ei = edge_index.astype(jnp.int32)
    src, dst = ei[0], ei[1]
    fwd = jnp.stack([src.reshape(N_CHUNKS, CHUNK), dst.reshape(N_CHUNKS, CHUNK)], 1)
    rev = fwd[:, ::-1]
    idx = jnp.stack([fwd, rev])
    zeros2 = jnp.zeros((ROWS_PER_TILE, D), jnp.float32)
    zeros1 = jnp.zeros((N_PAD,), jnp.float32)
    feat, deg = _sc_call(x, idx, zeros2, zeros1)
    aF = feat[0, :N_NODES]
    aR = feat[1, :N_NODES]
    return _tc_call(x, aF, deg[0], aR, deg[1],
                    W_l1.T, W_r1.T, W_l2.T, W_r2.T,
                    b_l1.reshape(1, D), b_l2.reshape(1, D))
